# baseline (speedup 1.0000x reference)
import functools

import jax
import jax.numpy as jnp
from jax import lax
from jax.experimental import pallas as pl
from jax.experimental.pallas import tpu as pltpu
from jax.experimental.pallas import tpu_sc as plsc

N = 10000
E = 320000
D_IN = 128
D_HID = 256
D_OUT = 128

W = 144
NC, NS = 2, 16
CHUNK = 128
CHUNK_G = 64
SLOTS = 2
N_ACC = 10112
ROWS_PER_TILE = N_ACC // NS
E_PAD = 327680
EDGES_PER_TILE = E_PAD // (NC * NS)
N_CHUNKS = EDGES_PER_TILE // CHUNK
ROUNDS = N_CHUNKS // SLOTS
N_CHUNKS_G = EDGES_PER_TILE // CHUNK_G
ROUNDS_G = N_CHUNKS_G // SLOTS

_HIGH = lax.Precision.HIGHEST


def _sc_mesh():
    return plsc.VectorSubcoreMesh(core_axis_name="c", subcore_axis_name="s")


def _sage_sc(x_pad, src, dst, zeros):
    @functools.partial(
        pl.kernel,
        out_type=jax.ShapeDtypeStruct((NC, N_ACC, W), jnp.float32),
        mesh=_sc_mesh(),
        compiler_params=pltpu.CompilerParams(use_tc_tiling_on_sc=False),
        scratch_types=[
            pltpu.VMEM_SHARED((N_ACC, W), jnp.float32),
            pltpu.VMEM((SLOTS, CHUNK), jnp.int32),
            pltpu.VMEM((SLOTS, CHUNK), jnp.int32),
            pltpu.VMEM((SLOTS, CHUNK, W), jnp.float32),
            pltpu.SemaphoreType.DMA((SLOTS,)),
            pltpu.SemaphoreType.DMA((SLOTS,)),
            pltpu.SemaphoreType.DMA((SLOTS,)),
        ],
    )
    def k(x_hbm, src_hbm, dst_hbm, zero_hbm, out_hbm,
          acc, src_v, dst_v, rows_v, sem_i, sem_g, sem_s):
        c = lax.axis_index("c")
        s = lax.axis_index("s")
        row0 = s * ROWS_PER_TILE
        pltpu.sync_copy(zero_hbm.at[pl.ds(row0, ROWS_PER_TILE)],
                        acc.at[pl.ds(row0, ROWS_PER_TILE)])
        plsc.subcore_barrier()
        tile_base = c * (E_PAD // NC) + s * EDGES_PER_TILE

        def idx_descs(X, base):
            return (pltpu.make_async_copy(
                        src_hbm.at[pl.ds(base, CHUNK)], src_v.at[X], sem_i.at[X]),
                    pltpu.make_async_copy(
                        dst_hbm.at[pl.ds(base, CHUNK)], dst_v.at[X], sem_i.at[X]))

        def gather_desc(X):
            return pltpu.make_async_copy(
                x_hbm.at[src_v.at[X]], rows_v.at[X], sem_g.at[X])

        def scatter_start(X):
            pltpu.async_copy(
                rows_v.at[X], acc.at[dst_v.at[X]], sem_s.at[X], add=True)

        def scatter_wait(X):
            pltpu.make_async_copy(
                rows_v.at[X], acc.at[dst_v.at[X]], sem_s.at[X]).wait()

        @pl.loop(0, ROUNDS)
        def _(j):
            base0 = tile_base + j * (SLOTS * CHUNK)
            for X in range(SLOTS):
                @pl.when(j > 0)
                def _():
                    scatter_wait(X)
                d1, d2 = idx_descs(X, base0 + X * CHUNK)
                d1.start()
                d2.start()
            for X in range(SLOTS):
                d1, d2 = idx_descs(X, base0 + X * CHUNK)
                d1.wait()
                d2.wait()
                gather_desc(X).start()
            for X in range(SLOTS):
                gather_desc(X).wait()
                scatter_start(X)

        for X in range(SLOTS):
            scatter_wait(X)
        plsc.subcore_barrier()
        pltpu.sync_copy(acc.at[pl.ds(row0, ROWS_PER_TILE)],
                        out_hbm.at[c].at[pl.ds(row0, ROWS_PER_TILE)])

    return k(x_pad, src, dst, zeros)


def _gat_sc(hw_pad, src, dst, a_src, a_dst, zeros):
    @functools.partial(
        pl.kernel,
        out_type=jax.ShapeDtypeStruct((NC, N_ACC, W), jnp.float32),
        mesh=_sc_mesh(),
        compiler_params=pltpu.CompilerParams(
            use_tc_tiling_on_sc=False, needs_layout_passes=False),
        scratch_types=[
            pltpu.VMEM_SHARED((N_ACC, W), jnp.float32),
            pltpu.VMEM((SLOTS, CHUNK_G), jnp.int32),
            pltpu.VMEM((SLOTS, CHUNK_G), jnp.int32),
            pltpu.VMEM((SLOTS, CHUNK_G, W), jnp.float32),
            pltpu.VMEM((SLOTS, CHUNK_G), jnp.float32),
            pltpu.VMEM((N_ACC,), jnp.float32),
            pltpu.VMEM((N_ACC,), jnp.float32),
            pltpu.SemaphoreType.DMA((SLOTS,)),
            pltpu.SemaphoreType.DMA((SLOTS,)),
            pltpu.SemaphoreType.DMA((SLOTS,)),
        ],
    )
    def k(hw_hbm, src_hbm, dst_hbm, asrc_hbm, adst_hbm, zero_hbm, out_hbm,
          acc, src_v, dst_v, rows_v, ex_v, asrc_v, adst_v,
          sem_i, sem_g, sem_s):
        c = lax.axis_index("c")
        s = lax.axis_index("s")
        row0 = s * ROWS_PER_TILE
        pltpu.sync_copy(zero_hbm.at[pl.ds(row0, ROWS_PER_TILE)],
                        acc.at[pl.ds(row0, ROWS_PER_TILE)])
        pltpu.sync_copy(asrc_hbm, asrc_v)
        pltpu.sync_copy(adst_hbm, adst_v)
        plsc.subcore_barrier()
        tile_base = c * (E_PAD // NC) + s * EDGES_PER_TILE

        def idx_descs(X, base):
            return (pltpu.make_async_copy(
                        src_hbm.at[pl.ds(base, CHUNK_G)], src_v.at[X], sem_i.at[X]),
                    pltpu.make_async_copy(
                        dst_hbm.at[pl.ds(base, CHUNK_G)], dst_v.at[X], sem_i.at[X]))

        def gather_desc(X):
            return pltpu.make_async_copy(
                hw_hbm.at[src_v.at[X]], rows_v.at[X], sem_g.at[X])

        def scatter_start(X):
            pltpu.async_copy(
                rows_v.at[X], acc.at[dst_v.at[X]], sem_s.at[X], add=True)

        def scatter_wait(X):
            pltpu.make_async_copy(
                rows_v.at[X], acc.at[dst_v.at[X]], sem_s.at[X]).wait()

        @pl.loop(0, ROUNDS_G)
        def _(j):
            base0 = tile_base + j * (SLOTS * CHUNK_G)
            for X in range(SLOTS):
                @pl.when(j > 0)
                def _():
                    scatter_wait(X)
                d1, d2 = idx_descs(X, base0 + X * CHUNK_G)
                d1.start()
                d2.start()
            for X in range(SLOTS):
                d1, d2 = idx_descs(X, base0 + X * CHUNK_G)
                d1.wait()
                d2.wait()
                gather_desc(X).start()
                for g in range(CHUNK_G // 16):
                    si = src_v[X, pl.ds(g * 16, 16)]
                    di = dst_v[X, pl.ds(g * 16, 16)]
                    av = plsc.load_gather(asrc_v, [si])
                    bv = plsc.load_gather(adst_v, [di])
                    e = av + bv
                    e = jnp.maximum(e, e * 0.2)
                    ex_v[X, pl.ds(g * 16, 16)] = jnp.exp(e)
            for X in range(SLOTS):
                gather_desc(X).wait()

                @pl.loop(0, CHUNK_G)
                def _(i):
                    spl = plsc.load_gather(
                        ex_v.at[X], [jnp.full((16,), i, jnp.int32)])
                    for cg in range(W // 16):
                        rows_v[X, i, pl.ds(cg * 16, 16)] = (
                            rows_v[X, i, pl.ds(cg * 16, 16)] * spl)

                scatter_start(X)

        for X in range(SLOTS):
            scatter_wait(X)
        plsc.subcore_barrier()
        pltpu.sync_copy(acc.at[pl.ds(row0, ROWS_PER_TILE)],
                        out_hbm.at[c].at[pl.ds(row0, ROWS_PER_TILE)])

    return k(hw_pad, src, dst, a_src, a_dst, zeros)


BR = 1000


def _tc_a(p0, p1, x, w_l, w_r, b_s, w_g, a_s, a_d, w_lin, b_lin):
    def body(p0_r, p1_r, x_r, wl_r, wr_r, bs_r, wg_r, as_r, ad_r, wlin_r,
             blin_r, hw_r, skip_r, aux_r):
        s = p0_r[...] + p1_r[...]
        agg = s[:, :D_IN]
        cnt = s[:, D_IN:D_IN + 1]
        mean = agg / jnp.maximum(cnt, 1.0)
        h = (jnp.dot(mean, wl_r[...], precision=_HIGH)
             + jnp.dot(x_r[...], wr_r[...], precision=_HIGH) + bs_r[...])
        h = jnp.maximum(h, 0.0)
        hw = jnp.dot(h, wg_r[...], precision=_HIGH)
        skip_r[...] = jnp.dot(h, wlin_r[...], precision=_HIGH) + blin_r[...]
        av = jnp.sum(hw * as_r[...], axis=1, keepdims=True)
        dv = jnp.sum(hw * ad_r[...], axis=1, keepdims=True)
        hw_r[...] = jnp.concatenate(
            [hw, jnp.ones((BR, W - D_OUT), jnp.float32)], axis=1)
        aux_r[...] = jnp.concatenate(
            [av, dv, jnp.zeros((BR, 126), jnp.float32)], axis=1)

    full = lambda shp: pl.BlockSpec(shp, lambda i: (0,) * len(shp))
    return pl.pallas_call(
        body,
        grid=(N // BR,),
        in_specs=[
            pl.BlockSpec((BR, W), lambda i: (i, 0)),
            pl.BlockSpec((BR, W), lambda i: (i, 0)),
            pl.BlockSpec((BR, D_IN), lambda i: (i, 0)),
            full((D_IN, D_HID)),
            full((D_IN, D_HID)),
            full((1, D_HID)),
            full((D_HID, D_OUT)),
            full((1, D_OUT)),
            full((1, D_OUT)),
            full((D_HID, D_OUT)),
            full((1, D_OUT)),
        ],
        out_specs=[
            pl.BlockSpec((BR, W), lambda i: (i, 0)),
            pl.BlockSpec((BR, D_OUT), lambda i: (i, 0)),
            pl.BlockSpec((BR, 128), lambda i: (i, 0)),
        ],
        out_shape=[
            jax.ShapeDtypeStruct((N, W), jnp.float32),
            jax.ShapeDtypeStruct((N, D_OUT), jnp.float32),
            jax.ShapeDtypeStruct((N, 128), jnp.float32),
        ],
    )(p0, p1, x, w_l, w_r, b_s, w_g, a_s, a_d, w_lin, b_lin)


def _tc_b(q0, q1, skip, b_g):
    def body(q0_r, q1_r, skip_r, bg_r, out_r):
        s = q0_r[...] + q1_r[...]
        denom = jnp.maximum(s[:, D_OUT:D_OUT + 1], 1e-16)
        out_r[...] = s[:, :D_OUT] / denom + bg_r[...] + skip_r[...]

    return pl.pallas_call(
        body,
        grid=(N // BR,),
        in_specs=[
            pl.BlockSpec((BR, W), lambda i: (i, 0)),
            pl.BlockSpec((BR, W), lambda i: (i, 0)),
            pl.BlockSpec((BR, D_OUT), lambda i: (i, 0)),
            pl.BlockSpec((1, D_OUT), lambda i: (0, 0)),
        ],
        out_specs=pl.BlockSpec((BR, D_OUT), lambda i: (i, 0)),
        out_shape=jax.ShapeDtypeStruct((N, D_OUT), jnp.float32),
    )(q0, q1, skip, b_g)


def kernel(x, edge_index, W_sage_l, W_sage_r, b_sage, W_gat, att_src,
           att_dst, b_gat, W_lin, b_lin):
    src = edge_index[0].astype(jnp.int32)
    dst = edge_index[1].astype(jnp.int32)
    pad = E_PAD - E
    src_p = jnp.concatenate([src, jnp.zeros((pad,), jnp.int32)])
    dst_p = jnp.concatenate([dst, jnp.full((pad,), N, jnp.int32)])
    x_pad = jnp.concatenate([x, jnp.ones((N, W - D_IN), jnp.float32)], axis=1)
    zeros = jnp.zeros((N_ACC, W), jnp.float32)

    p = _sage_sc(x_pad, src_p, dst_p, zeros)
    hw_pad, skip, aux = _tc_a(
        p[0, :N], p[1, :N], x, W_sage_l, W_sage_r, b_sage.reshape(1, -1),
        W_gat, att_src.reshape(1, -1), att_dst.reshape(1, -1), W_lin,
        b_lin.reshape(1, -1))
    a_s = jnp.pad(aux[:, 0], (0, N_ACC - N))
    a_d = jnp.pad(aux[:, 1], (0, N_ACC - N))
    q = _gat_sc(hw_pad, src_p, dst_p, a_s, a_d, zeros)
    out = _tc_b(q[0, :N], q[1, :N], skip, b_gat.reshape(1, -1))
    return out


# speedup vs baseline: 2.1490x; 2.1490x over previous
import functools

import jax
import jax.numpy as jnp
from jax import lax
from jax.experimental import pallas as pl
from jax.experimental.pallas import tpu as pltpu
from jax.experimental.pallas import tpu_sc as plsc

N = 10000
E = 320000
D_IN = 128
D_HID = 256
D_OUT = 128

W = 144
NC, NS = 2, 16
NW = NC * NS
CHUNK = 128
CHUNK_G = 64
SLOTS = 2
N_ACC = 10112
ROWS_PER_TILE = N_ACC // NS

RND_S_BIG, N_BIG_S = 40, 2
RND_S = 39
RND_G_BIG, N_BIG_G = 79, 4
RND_G = 78

_HIGH = lax.Precision.HIGHEST


def _sc_mesh():
    return plsc.VectorSubcoreMesh(core_axis_name="c", subcore_axis_name="s")


def _tile_plan(c, s, rounds_small, rounds_big, n_big, edges_per_round):
    w = c * NS + s
    extra = rounds_big - rounds_small
    nb = jnp.minimum(w, n_big)
    base = (w * rounds_small + nb * extra) * edges_per_round
    rounds = jnp.where(w < n_big, rounds_big, rounds_small)
    return base, rounds


def _sage_sc(x_pad, edges, zeros):
    @functools.partial(
        pl.kernel,
        out_type=jax.ShapeDtypeStruct((NC, N_ACC, W), jnp.float32),
        mesh=_sc_mesh(),
        compiler_params=pltpu.CompilerParams(use_tc_tiling_on_sc=False),
        scratch_types=[
            pltpu.VMEM_SHARED((N_ACC, W), jnp.float32),
            pltpu.VMEM((SLOTS, CHUNK), jnp.int32),
            pltpu.VMEM((SLOTS, CHUNK), jnp.int32),
            pltpu.VMEM((SLOTS, CHUNK, W), jnp.float32),
            pltpu.SemaphoreType.DMA((SLOTS,)),
            pltpu.SemaphoreType.DMA((SLOTS,)),
            pltpu.SemaphoreType.DMA((SLOTS,)),
        ],
    )
    def k(x_hbm, e_hbm, zero_hbm, out_hbm,
          acc, src_v, dst_v, rows_v, sem_i, sem_g, sem_s):
        c = lax.axis_index("c")
        s = lax.axis_index("s")
        row0 = s * ROWS_PER_TILE
        pltpu.sync_copy(zero_hbm.at[pl.ds(row0, ROWS_PER_TILE)],
                        acc.at[pl.ds(row0, ROWS_PER_TILE)])
        plsc.subcore_barrier()
        tile_base, rounds = _tile_plan(c, s, RND_S, RND_S_BIG, N_BIG_S,
                                       SLOTS * CHUNK)

        def idx_descs(X, base):
            return (pltpu.make_async_copy(
                        e_hbm.at[0].at[pl.ds(base, CHUNK)], src_v.at[X],
                        sem_i.at[X]),
                    pltpu.make_async_copy(
                        e_hbm.at[1].at[pl.ds(base, CHUNK)], dst_v.at[X],
                        sem_i.at[X]))

        def gather_desc(X):
            return pltpu.make_async_copy(
                x_hbm.at[src_v.at[X]], rows_v.at[X], sem_g.at[X])

        def scatter_start(X):
            pltpu.async_copy(
                rows_v.at[X], acc.at[dst_v.at[X]], sem_s.at[X], add=True)

        def scatter_wait(X):
            pltpu.make_async_copy(
                rows_v.at[X], acc.at[dst_v.at[X]], sem_s.at[X]).wait()

        @pl.loop(0, rounds)
        def _(j):
            base0 = tile_base + j * (SLOTS * CHUNK)
            for X in range(SLOTS):
                @pl.when(j > 0)
                def _():
                    scatter_wait(X)
                d1, d2 = idx_descs(X, base0 + X * CHUNK)
                d1.start()
                d2.start()
            for X in range(SLOTS):
                d1, d2 = idx_descs(X, base0 + X * CHUNK)
                d1.wait()
                d2.wait()
                gather_desc(X).start()
            for X in range(SLOTS):
                gather_desc(X).wait()
                scatter_start(X)

        for X in range(SLOTS):
            scatter_wait(X)
        plsc.subcore_barrier()
        pltpu.sync_copy(acc.at[pl.ds(row0, ROWS_PER_TILE)],
                        out_hbm.at[c].at[pl.ds(row0, ROWS_PER_TILE)])

    return k(x_pad, edges, zeros)


def _gat_sc(hw_pad, edges, aux, zeros):
    @functools.partial(
        pl.kernel,
        out_type=jax.ShapeDtypeStruct((NC, N_ACC, W), jnp.float32),
        mesh=_sc_mesh(),
        compiler_params=pltpu.CompilerParams(
            use_tc_tiling_on_sc=False, needs_layout_passes=False),
        scratch_types=[
            pltpu.VMEM_SHARED((N_ACC, W), jnp.float32),
            pltpu.VMEM((SLOTS, CHUNK_G), jnp.int32),
            pltpu.VMEM((SLOTS, CHUNK_G), jnp.int32),
            pltpu.VMEM((SLOTS, CHUNK_G, W), jnp.float32),
            pltpu.VMEM((SLOTS, CHUNK_G), jnp.float32),
            pltpu.VMEM((N_ACC,), jnp.float32),
            pltpu.VMEM((N_ACC,), jnp.float32),
            pltpu.SemaphoreType.DMA((SLOTS,)),
            pltpu.SemaphoreType.DMA((SLOTS,)),
            pltpu.SemaphoreType.DMA((SLOTS,)),
        ],
    )
    def k(hw_hbm, e_hbm, aux_hbm, zero_hbm, out_hbm,
          acc, src_v, dst_v, rows_v, ex_v, asrc_v, adst_v,
          sem_i, sem_g, sem_s):
        c = lax.axis_index("c")
        s = lax.axis_index("s")
        row0 = s * ROWS_PER_TILE
        pltpu.sync_copy(zero_hbm.at[pl.ds(row0, ROWS_PER_TILE)],
                        acc.at[pl.ds(row0, ROWS_PER_TILE)])
        pltpu.sync_copy(aux_hbm.at[0], asrc_v)
        pltpu.sync_copy(aux_hbm.at[1], adst_v)
        plsc.subcore_barrier()
        tile_base, rounds = _tile_plan(c, s, RND_G, RND_G_BIG, N_BIG_G,
                                       SLOTS * CHUNK_G)

        def idx_descs(X, base):
            return (pltpu.make_async_copy(
                        e_hbm.at[0].at[pl.ds(base, CHUNK_G)], src_v.at[X],
                        sem_i.at[X]),
                    pltpu.make_async_copy(
                        e_hbm.at[1].at[pl.ds(base, CHUNK_G)], dst_v.at[X],
                        sem_i.at[X]))

        def gather_desc(X):
            return pltpu.make_async_copy(
                hw_hbm.at[src_v.at[X]], rows_v.at[X], sem_g.at[X])

        def scatter_start(X):
            pltpu.async_copy(
                rows_v.at[X], acc.at[dst_v.at[X]], sem_s.at[X], add=True)

        def scatter_wait(X):
            pltpu.make_async_copy(
                rows_v.at[X], acc.at[dst_v.at[X]], sem_s.at[X]).wait()

        @pl.loop(0, rounds)
        def _(j):
            base0 = tile_base + j * (SLOTS * CHUNK_G)
            for X in range(SLOTS):
                @pl.when(j > 0)
                def _():
                    scatter_wait(X)
                d1, d2 = idx_descs(X, base0 + X * CHUNK_G)
                d1.start()
                d2.start()
            for X in range(SLOTS):
                d1, d2 = idx_descs(X, base0 + X * CHUNK_G)
                d1.wait()
                d2.wait()
                gather_desc(X).start()
                for g in range(CHUNK_G // 16):
                    si = src_v[X, pl.ds(g * 16, 16)]
                    di = dst_v[X, pl.ds(g * 16, 16)]
                    av = plsc.load_gather(asrc_v, [si])
                    bv = plsc.load_gather(adst_v, [di])
                    e = av + bv
                    e = jnp.maximum(e, e * 0.2)
                    ex_v[X, pl.ds(g * 16, 16)] = jnp.exp(e)
            for X in range(SLOTS):
                gather_desc(X).wait()

                @pl.loop(0, CHUNK_G)
                def _(i):
                    spl = plsc.load_gather(
                        ex_v.at[X], [jnp.full((16,), i, jnp.int32)])
                    for cg in range(W // 16):
                        rows_v[X, i, pl.ds(cg * 16, 16)] = (
                            rows_v[X, i, pl.ds(cg * 16, 16)] * spl)

                scatter_start(X)

        for X in range(SLOTS):
            scatter_wait(X)
        plsc.subcore_barrier()
        pltpu.sync_copy(acc.at[pl.ds(row0, ROWS_PER_TILE)],
                        out_hbm.at[c].at[pl.ds(row0, ROWS_PER_TILE)])

    return k(hw_pad, edges, aux, zeros)


BR = 1024


def _tc_a(p, x, w_l, w_r, b_s, w_g, a_s, a_d, w_lin, b_lin):
    def body(p0_r, p1_r, x_r, wl_r, wr_r, bs_r, wg_r, as_r, ad_r, wlin_r,
             blin_r, hw_r, skip_r, aux_r):
        s = p0_r[...] + p1_r[...]
        agg = s[:, :D_IN]
        cnt = s[:, D_IN:D_IN + 1]
        mean = agg / jnp.maximum(cnt, 1.0)
        h = (jnp.dot(mean, wl_r[...], precision=_HIGH)
             + jnp.dot(x_r[...], wr_r[...], precision=_HIGH) + bs_r[...])
        h = jnp.maximum(h, 0.0)
        hw = jnp.dot(h, wg_r[...], precision=_HIGH)
        skip_r[...] = jnp.dot(h, wlin_r[...], precision=_HIGH) + blin_r[...]
        av = lax.dot_general(as_r[...], hw, (((1,), (1,)), ((), ())),
                             precision=_HIGH)
        dv = lax.dot_general(ad_r[...], hw, (((1,), (1,)), ((), ())),
                             precision=_HIGH)
        hw_r[...] = jnp.concatenate(
            [hw, jnp.ones((BR, W - D_OUT), jnp.float32)], axis=1)
        aux_r[...] = jnp.concatenate(
            [av, dv, jnp.zeros((14, BR), jnp.float32)], axis=0)

    full = lambda shp: pl.BlockSpec(shp, lambda i: (0,) * len(shp))
    return pl.pallas_call(
        body,
        grid=(pl.cdiv(N, BR),),
        in_specs=[
            pl.BlockSpec((None, BR, W), lambda i: (0, i, 0)),
            pl.BlockSpec((None, BR, W), lambda i: (1, i, 0)),
            pl.BlockSpec((BR, D_IN), lambda i: (i, 0)),
            full((D_IN, D_HID)),
            full((D_IN, D_HID)),
            full((1, D_HID)),
            full((D_HID, D_OUT)),
            full((1, D_OUT)),
            full((1, D_OUT)),
            full((D_HID, D_OUT)),
            full((1, D_OUT)),
        ],
        out_specs=[
            pl.BlockSpec((BR, W), lambda i: (i, 0)),
            pl.BlockSpec((BR, D_OUT), lambda i: (i, 0)),
            pl.BlockSpec((16, BR), lambda i: (0, i)),
        ],
        out_shape=[
            jax.ShapeDtypeStruct((N, W), jnp.float32),
            jax.ShapeDtypeStruct((N, D_OUT), jnp.float32),
            jax.ShapeDtypeStruct((16, N_ACC), jnp.float32),
        ],
    )(p, p, x, w_l, w_r, b_s, w_g, a_s, a_d, w_lin, b_lin)


def _tc_b(q, skip, b_g):
    def body(q0_r, q1_r, skip_r, bg_r, out_r):
        s = q0_r[...] + q1_r[...]
        denom = jnp.maximum(s[:, D_OUT:D_OUT + 1], 1e-16)
        out_r[...] = s[:, :D_OUT] / denom + bg_r[...] + skip_r[...]

    return pl.pallas_call(
        body,
        grid=(pl.cdiv(N, BR),),
        in_specs=[
            pl.BlockSpec((None, BR, W), lambda i: (0, i, 0)),
            pl.BlockSpec((None, BR, W), lambda i: (1, i, 0)),
            pl.BlockSpec((BR, D_OUT), lambda i: (i, 0)),
            pl.BlockSpec((1, D_OUT), lambda i: (0, 0)),
        ],
        out_specs=pl.BlockSpec((BR, D_OUT), lambda i: (i, 0)),
        out_shape=jax.ShapeDtypeStruct((N, D_OUT), jnp.float32),
    )(q, q, skip, b_g)


def kernel(x, edge_index, W_sage_l, W_sage_r, b_sage, W_gat, att_src,
           att_dst, b_gat, W_lin, b_lin):
    edges = edge_index.astype(jnp.int32)
    x_pad = jnp.concatenate([x, jnp.ones((N, W - D_IN), jnp.float32)], axis=1)
    zeros = jnp.zeros((N_ACC, W), jnp.float32)

    p = _sage_sc(x_pad, edges, zeros)
    hw_pad, skip, aux = _tc_a(
        p, x, W_sage_l, W_sage_r, b_sage.reshape(1, -1),
        W_gat, att_src.reshape(1, -1), att_dst.reshape(1, -1), W_lin,
        b_lin.reshape(1, -1))
    q = _gat_sc(hw_pad, edges, aux, zeros)
    out = _tc_b(q, skip, b_gat.reshape(1, -1))
    return out


# speedup vs baseline: 2.5090x; 1.1675x over previous
import functools

import jax
import jax.numpy as jnp
from jax import lax
from jax.experimental import pallas as pl
from jax.experimental.pallas import tpu as pltpu
from jax.experimental.pallas import tpu_sc as plsc

N = 10000
E = 320000
D_IN = 128
D_HID = 256
D_OUT = 128

W = 144
NC, NS = 2, 16
NW = NC * NS
CHUNK = 128
CHUNK_G = 64
SLOTS = 2
N_ACC = 10112
ROWS_PER_TILE = N_ACC // NS

RND_S_BIG, N_BIG_S = 40, 2
RND_S = 39
RND_G_BIG, N_BIG_G = 79, 4
RND_G = 78

_HIGH = lax.Precision.DEFAULT


def _sc_mesh():
    return plsc.VectorSubcoreMesh(core_axis_name="c", subcore_axis_name="s")


def _tile_plan(c, s, rounds_small, rounds_big, n_big, edges_per_round):
    w = c * NS + s
    extra = rounds_big - rounds_small
    nb = jnp.minimum(w, n_big)
    base = (w * rounds_small + nb * extra) * edges_per_round
    rounds = jnp.where(w < n_big, rounds_big, rounds_small)
    return base, rounds


def _sage_sc(x_pad, edges, zeros):
    @functools.partial(
        pl.kernel,
        out_type=jax.ShapeDtypeStruct((NC, N_ACC, W), jnp.float32),
        mesh=_sc_mesh(),
        compiler_params=pltpu.CompilerParams(use_tc_tiling_on_sc=False),
        scratch_types=[
            pltpu.VMEM_SHARED((N_ACC, W), jnp.float32),
            pltpu.VMEM((SLOTS, CHUNK), jnp.int32),
            pltpu.VMEM((SLOTS, CHUNK), jnp.int32),
            pltpu.VMEM((SLOTS, CHUNK, W), jnp.float32),
            pltpu.SemaphoreType.DMA((SLOTS,)),
            pltpu.SemaphoreType.DMA((SLOTS,)),
            pltpu.SemaphoreType.DMA((SLOTS,)),
        ],
    )
    def k(x_hbm, e_hbm, zero_hbm, out_hbm,
          acc, src_v, dst_v, rows_v, sem_i, sem_g, sem_s):
        c = lax.axis_index("c")
        s = lax.axis_index("s")
        row0 = s * ROWS_PER_TILE
        pltpu.sync_copy(zero_hbm.at[pl.ds(row0, ROWS_PER_TILE)],
                        acc.at[pl.ds(row0, ROWS_PER_TILE)])
        plsc.subcore_barrier()
        tile_base, rounds = _tile_plan(c, s, RND_S, RND_S_BIG, N_BIG_S,
                                       SLOTS * CHUNK)

        def idx_descs(X, base):
            return (pltpu.make_async_copy(
                        e_hbm.at[0].at[pl.ds(base, CHUNK)], src_v.at[X],
                        sem_i.at[X]),
                    pltpu.make_async_copy(
                        e_hbm.at[1].at[pl.ds(base, CHUNK)], dst_v.at[X],
                        sem_i.at[X]))

        def gather_desc(X):
            return pltpu.make_async_copy(
                x_hbm.at[src_v.at[X]], rows_v.at[X], sem_g.at[X])

        def scatter_start(X):
            pltpu.async_copy(
                rows_v.at[X], acc.at[dst_v.at[X]], sem_s.at[X], add=True)

        def scatter_wait(X):
            pltpu.make_async_copy(
                rows_v.at[X], acc.at[dst_v.at[X]], sem_s.at[X]).wait()

        @pl.loop(0, rounds)
        def _(j):
            base0 = tile_base + j * (SLOTS * CHUNK)
            for X in range(SLOTS):
                @pl.when(j > 0)
                def _():
                    scatter_wait(X)
                d1, d2 = idx_descs(X, base0 + X * CHUNK)
                d1.start()
                d2.start()
            for X in range(SLOTS):
                d1, d2 = idx_descs(X, base0 + X * CHUNK)
                d1.wait()
                d2.wait()
                gather_desc(X).start()
            for X in range(SLOTS):
                gather_desc(X).wait()
                scatter_start(X)

        for X in range(SLOTS):
            scatter_wait(X)
        plsc.subcore_barrier()
        pltpu.sync_copy(acc.at[pl.ds(row0, ROWS_PER_TILE)],
                        out_hbm.at[c].at[pl.ds(row0, ROWS_PER_TILE)])

    return k(x_pad, edges, zeros)


def _gat_sc(hw_pad, edges, aux, zeros):
    @functools.partial(
        pl.kernel,
        out_type=jax.ShapeDtypeStruct((NC, N_ACC, W), jnp.float32),
        mesh=_sc_mesh(),
        compiler_params=pltpu.CompilerParams(
            use_tc_tiling_on_sc=False, needs_layout_passes=False),
        scratch_types=[
            pltpu.VMEM_SHARED((N_ACC, W), jnp.float32),
            pltpu.VMEM((SLOTS, CHUNK_G), jnp.int32),
            pltpu.VMEM((SLOTS, CHUNK_G), jnp.int32),
            pltpu.VMEM((SLOTS, CHUNK_G, W), jnp.float32),
            pltpu.VMEM((SLOTS, CHUNK_G), jnp.float32),
            pltpu.VMEM((N_ACC,), jnp.float32),
            pltpu.VMEM((N_ACC,), jnp.float32),
            pltpu.SemaphoreType.DMA((SLOTS,)),
            pltpu.SemaphoreType.DMA((SLOTS,)),
            pltpu.SemaphoreType.DMA((SLOTS,)),
        ],
    )
    def k(hw_hbm, e_hbm, aux_hbm, zero_hbm, out_hbm,
          acc, src_v, dst_v, rows_v, ex_v, asrc_v, adst_v,
          sem_i, sem_g, sem_s):
        c = lax.axis_index("c")
        s = lax.axis_index("s")
        row0 = s * ROWS_PER_TILE
        pltpu.sync_copy(zero_hbm.at[pl.ds(row0, ROWS_PER_TILE)],
                        acc.at[pl.ds(row0, ROWS_PER_TILE)])
        pltpu.sync_copy(aux_hbm.at[0], asrc_v)
        pltpu.sync_copy(aux_hbm.at[1], adst_v)
        plsc.subcore_barrier()
        tile_base, rounds = _tile_plan(c, s, RND_G, RND_G_BIG, N_BIG_G,
                                       SLOTS * CHUNK_G)

        def idx_descs(X, base):
            return (pltpu.make_async_copy(
                        e_hbm.at[0].at[pl.ds(base, CHUNK_G)], src_v.at[X],
                        sem_i.at[X]),
                    pltpu.make_async_copy(
                        e_hbm.at[1].at[pl.ds(base, CHUNK_G)], dst_v.at[X],
                        sem_i.at[X]))

        def gather_desc(X):
            return pltpu.make_async_copy(
                hw_hbm.at[src_v.at[X]], rows_v.at[X], sem_g.at[X])

        def scatter_start(X):
            pltpu.async_copy(
                rows_v.at[X], acc.at[dst_v.at[X]], sem_s.at[X], add=True)

        def scatter_wait(X):
            pltpu.make_async_copy(
                rows_v.at[X], acc.at[dst_v.at[X]], sem_s.at[X]).wait()

        @pl.loop(0, rounds)
        def _(j):
            base0 = tile_base + j * (SLOTS * CHUNK_G)
            for X in range(SLOTS):
                @pl.when(j > 0)
                def _():
                    scatter_wait(X)
                d1, d2 = idx_descs(X, base0 + X * CHUNK_G)
                d1.start()
                d2.start()
            for X in range(SLOTS):
                d1, d2 = idx_descs(X, base0 + X * CHUNK_G)
                d1.wait()
                d2.wait()
                gather_desc(X).start()
                for g in range(CHUNK_G // 16):
                    si = src_v[X, pl.ds(g * 16, 16)]
                    di = dst_v[X, pl.ds(g * 16, 16)]
                    av = plsc.load_gather(asrc_v, [si])
                    bv = plsc.load_gather(adst_v, [di])
                    e = av + bv
                    e = jnp.maximum(e, e * 0.2)
                    ex_v[X, pl.ds(g * 16, 16)] = jnp.exp(e)
            for X in range(SLOTS):
                gather_desc(X).wait()

                @plsc.parallel_loop(0, CHUNK_G, 1, unroll=4)
                def _(i):
                    spl = plsc.load_gather(
                        ex_v.at[X], [jnp.full((16,), i, jnp.int32)])
                    for cg in range(D_OUT // 16):
                        rows_v[X, i, pl.ds(cg * 16, 16)] = (
                            rows_v[X, i, pl.ds(cg * 16, 16)] * spl)
                    rows_v[X, i, pl.ds(D_OUT, 16)] = spl

                scatter_start(X)

        for X in range(SLOTS):
            scatter_wait(X)
        plsc.subcore_barrier()
        pltpu.sync_copy(acc.at[pl.ds(row0, ROWS_PER_TILE)],
                        out_hbm.at[c].at[pl.ds(row0, ROWS_PER_TILE)])

    return k(hw_pad, edges, aux, zeros)


BR = 1024


def _tc_a(p, x, w_1, b_s, w_2, a_s, a_d, b_lin):
    def body(p0_r, p1_r, x_r, w1_r, bs_r, w2_r, as_r, ad_r,
             blin_r, hw_r, skip_r, aux_r):
        s = p0_r[...] + p1_r[...]
        agg = s[:, :D_IN]
        cnt = s[:, D_IN:D_IN + 1]
        mean = agg / jnp.maximum(cnt, 1.0)
        lhs = jnp.concatenate([mean, x_r[...]], axis=1)
        h = jnp.dot(lhs, w1_r[...], precision=_HIGH) + bs_r[...]
        h = jnp.maximum(h, 0.0)
        hs = jnp.dot(h, w2_r[...], precision=_HIGH)
        hw = hs[:, :D_OUT]
        skip_r[...] = hs[:, D_OUT:] + blin_r[...]
        av = lax.dot_general(as_r[...], hw, (((1,), (1,)), ((), ())),
                             precision=_HIGH)
        dv = lax.dot_general(ad_r[...], hw, (((1,), (1,)), ((), ())),
                             precision=_HIGH)
        hw_r[...] = jnp.concatenate(
            [hw, jnp.ones((BR, W - D_OUT), jnp.float32)], axis=1)
        aux_r[...] = jnp.concatenate(
            [av, dv, jnp.zeros((14, BR), jnp.float32)], axis=0)

    full = lambda shp: pl.BlockSpec(shp, lambda i: (0,) * len(shp))
    return pl.pallas_call(
        body,
        grid=(pl.cdiv(N, BR),),
        in_specs=[
            pl.BlockSpec((None, BR, W), lambda i: (0, i, 0)),
            pl.BlockSpec((None, BR, W), lambda i: (1, i, 0)),
            pl.BlockSpec((BR, D_IN), lambda i: (i, 0)),
            full((2 * D_IN, D_HID)),
            full((1, D_HID)),
            full((D_HID, 2 * D_OUT)),
            full((1, D_OUT)),
            full((1, D_OUT)),
            full((1, D_OUT)),
        ],
        out_specs=[
            pl.BlockSpec((BR, W), lambda i: (i, 0)),
            pl.BlockSpec((BR, D_OUT), lambda i: (i, 0)),
            pl.BlockSpec((16, BR), lambda i: (0, i)),
        ],
        out_shape=[
            jax.ShapeDtypeStruct((N, W), jnp.float32),
            jax.ShapeDtypeStruct((N, D_OUT), jnp.float32),
            jax.ShapeDtypeStruct((16, N_ACC), jnp.float32),
        ],
    )(p, p, x, w_1, b_s, w_2, a_s, a_d, b_lin)


def _tc_b(q, skip, b_g):
    def body(q0_r, q1_r, skip_r, bg_r, out_r):
        s = q0_r[...] + q1_r[...]
        denom = jnp.maximum(s[:, D_OUT:D_OUT + 1], 1e-16)
        out_r[...] = s[:, :D_OUT] / denom + bg_r[...] + skip_r[...]

    return pl.pallas_call(
        body,
        grid=(pl.cdiv(N, BR),),
        in_specs=[
            pl.BlockSpec((None, BR, W), lambda i: (0, i, 0)),
            pl.BlockSpec((None, BR, W), lambda i: (1, i, 0)),
            pl.BlockSpec((BR, D_OUT), lambda i: (i, 0)),
            pl.BlockSpec((1, D_OUT), lambda i: (0, 0)),
        ],
        out_specs=pl.BlockSpec((BR, D_OUT), lambda i: (i, 0)),
        out_shape=jax.ShapeDtypeStruct((N, D_OUT), jnp.float32),
    )(q, q, skip, b_g)


def kernel(x, edge_index, W_sage_l, W_sage_r, b_sage, W_gat, att_src,
           att_dst, b_gat, W_lin, b_lin):
    edges = edge_index.astype(jnp.int32)
    x_pad = jnp.concatenate([x, jnp.ones((N, W - D_IN), jnp.float32)], axis=1)
    zeros = jnp.zeros((N_ACC, W), jnp.float32)

    w_1 = jnp.concatenate([W_sage_l, W_sage_r], axis=0)
    w_2 = jnp.concatenate([W_gat, W_lin], axis=1)
    p = _sage_sc(x_pad, edges, zeros)
    hw_pad, skip, aux = _tc_a(
        p, x, w_1, b_sage.reshape(1, -1), w_2,
        att_src.reshape(1, -1), att_dst.reshape(1, -1),
        b_lin.reshape(1, -1))
    q = _gat_sc(hw_pad, edges, aux, zeros)
    out = _tc_b(q, skip, b_gat.reshape(1, -1))
    return out


# speedup vs baseline: 2.9759x; 1.1861x over previous
import functools

import jax
import jax.numpy as jnp
from jax import lax
from jax.experimental import pallas as pl
from jax.experimental.pallas import tpu as pltpu
from jax.experimental.pallas import tpu_sc as plsc

N = 10000
E = 320000
D_IN = 128
D_HID = 256
D_OUT = 128

NC, NS = 2, 16
CHUNK = 128
CHUNK_G = 64
SLOTS = 2
N_ACC = 10112
ROWS_PER_TILE = N_ACC // NS

RND_S_BIG, N_BIG_S = 40, 2
RND_S = 39
RND_G_BIG, N_BIG_G = 79, 4
RND_G = 78


def _sc_mesh():
    return plsc.VectorSubcoreMesh(core_axis_name="c", subcore_axis_name="s")


def _tile_plan(c, s, rounds_small, rounds_big, n_big, edges_per_round):
    w = c * NS + s
    extra = rounds_big - rounds_small
    nb = jnp.minimum(w, n_big)
    base = (w * rounds_small + nb * extra) * edges_per_round
    rounds = jnp.where(w < n_big, rounds_big, rounds_small)
    return base, rounds


_SC_OUT = (jax.ShapeDtypeStruct((NC, N_ACC, D_IN), jnp.float32),
           jax.ShapeDtypeStruct((NC, N_ACC, 16), jnp.float32))


def _sage_sc(x, e_flat, zeros, zeros16):
    @functools.partial(
        pl.kernel,
        out_type=_SC_OUT,
        mesh=_sc_mesh(),
        compiler_params=pltpu.CompilerParams(use_tc_tiling_on_sc=False),
        scratch_types=[
            pltpu.VMEM_SHARED((N_ACC, D_IN), jnp.float32),
            pltpu.VMEM_SHARED((N_ACC, 16), jnp.float32),
            pltpu.VMEM((SLOTS, CHUNK), jnp.int32),
            pltpu.VMEM((SLOTS, CHUNK), jnp.int32),
            pltpu.VMEM((SLOTS, CHUNK, D_IN), jnp.float32),
            pltpu.VMEM((CHUNK, 16), jnp.float32),
            pltpu.SemaphoreType.DMA((SLOTS,)),
            pltpu.SemaphoreType.DMA((SLOTS,)),
            pltpu.SemaphoreType.DMA((SLOTS,)),
            pltpu.SemaphoreType.DMA((SLOTS,)),
        ],
    )
    def k(x_hbm, e_hbm, zero_hbm, zero16_hbm, out_hbm, cnt_hbm,
          acc, acc2, src_v, dst_v, rows_v, ones_v, sem_i, sem_g, sem_s,
          sem_c):
        c = lax.axis_index("c")
        s = lax.axis_index("s")
        row0 = s * ROWS_PER_TILE
        pltpu.sync_copy(zero_hbm.at[pl.ds(row0, ROWS_PER_TILE)],
                        acc.at[pl.ds(row0, ROWS_PER_TILE)])
        pltpu.sync_copy(zero16_hbm.at[pl.ds(row0, ROWS_PER_TILE)],
                        acc2.at[pl.ds(row0, ROWS_PER_TILE)])

        @pl.loop(0, CHUNK)
        def _(i):
            ones_v[i, :] = jnp.full((16,), 1.0, jnp.float32)

        plsc.subcore_barrier()
        tile_base, rounds = _tile_plan(c, s, RND_S, RND_S_BIG, N_BIG_S,
                                       SLOTS * CHUNK)

        def idx_descs(X, base):
            return (pltpu.make_async_copy(
                        e_hbm.at[pl.ds(base, CHUNK)], src_v.at[X],
                        sem_i.at[X]),
                    pltpu.make_async_copy(
                        e_hbm.at[pl.ds(E + base, CHUNK)], dst_v.at[X],
                        sem_i.at[X]))

        def gather_desc(X):
            return pltpu.make_async_copy(
                x_hbm.at[src_v.at[X]], rows_v.at[X], sem_g.at[X])

        def scatter_start(X):
            pltpu.async_copy(
                rows_v.at[X], acc.at[dst_v.at[X]], sem_s.at[X], add=True)
            pltpu.async_copy(
                ones_v, acc2.at[dst_v.at[X]], sem_c.at[X], add=True)

        def scatter_wait(X):
            pltpu.make_async_copy(
                rows_v.at[X], acc.at[dst_v.at[X]], sem_s.at[X]).wait()
            pltpu.make_async_copy(
                ones_v, acc2.at[dst_v.at[X]], sem_c.at[X]).wait()

        @pl.loop(0, rounds)
        def _(j):
            base0 = tile_base + j * (SLOTS * CHUNK)
            for X in range(SLOTS):
                @pl.when(j > 0)
                def _():
                    scatter_wait(X)
                d1, d2 = idx_descs(X, base0 + X * CHUNK)
                d1.start()
                d2.start()
            for X in range(SLOTS):
                d1, d2 = idx_descs(X, base0 + X * CHUNK)
                d1.wait()
                d2.wait()
                gather_desc(X).start()
            for X in range(SLOTS):
                gather_desc(X).wait()
                scatter_start(X)

        for X in range(SLOTS):
            scatter_wait(X)
        plsc.subcore_barrier()
        pltpu.sync_copy(acc.at[pl.ds(row0, ROWS_PER_TILE)],
                        out_hbm.at[c].at[pl.ds(row0, ROWS_PER_TILE)])
        pltpu.sync_copy(acc2.at[pl.ds(row0, ROWS_PER_TILE)],
                        cnt_hbm.at[c].at[pl.ds(row0, ROWS_PER_TILE)])

    return k(x, e_flat, zeros, zeros16)


def _gat_sc(hw, e_flat, aux, zeros, zeros16):
    @functools.partial(
        pl.kernel,
        out_type=_SC_OUT,
        mesh=_sc_mesh(),
        compiler_params=pltpu.CompilerParams(
            use_tc_tiling_on_sc=False, needs_layout_passes=False),
        scratch_types=[
            pltpu.VMEM_SHARED((N_ACC, D_OUT), jnp.float32),
            pltpu.VMEM_SHARED((N_ACC, 16), jnp.float32),
            pltpu.VMEM((SLOTS, CHUNK_G), jnp.int32),
            pltpu.VMEM((SLOTS, CHUNK_G), jnp.int32),
            pltpu.VMEM((SLOTS, CHUNK_G, D_OUT), jnp.float32),
            pltpu.VMEM((SLOTS, CHUNK_G, 16), jnp.float32),
            pltpu.VMEM((SLOTS, CHUNK_G), jnp.float32),
            pltpu.VMEM((N_ACC,), jnp.float32),
            pltpu.VMEM((N_ACC,), jnp.float32),
            pltpu.SemaphoreType.DMA((SLOTS,)),
            pltpu.SemaphoreType.DMA((SLOTS,)),
            pltpu.SemaphoreType.DMA((SLOTS,)),
            pltpu.SemaphoreType.DMA((SLOTS,)),
        ],
    )
    def k(hw_hbm, e_hbm, aux_hbm, zero_hbm, zero16_hbm, out_hbm, den_hbm,
          acc, acc2, src_v, dst_v, rows_v, exrow_v, ex_v, asrc_v, adst_v,
          sem_i, sem_g, sem_s, sem_c):
        c = lax.axis_index("c")
        s = lax.axis_index("s")
        row0 = s * ROWS_PER_TILE
        pltpu.sync_copy(zero_hbm.at[pl.ds(row0, ROWS_PER_TILE)],
                        acc.at[pl.ds(row0, ROWS_PER_TILE)])
        pltpu.sync_copy(zero16_hbm.at[pl.ds(row0, ROWS_PER_TILE)],
                        acc2.at[pl.ds(row0, ROWS_PER_TILE)])
        pltpu.sync_copy(aux_hbm.at[0], asrc_v)
        pltpu.sync_copy(aux_hbm.at[1], adst_v)
        plsc.subcore_barrier()
        tile_base, rounds = _tile_plan(c, s, RND_G, RND_G_BIG, N_BIG_G,
                                       SLOTS * CHUNK_G)

        def idx_descs(X, base):
            return (pltpu.make_async_copy(
                        e_hbm.at[pl.ds(base, CHUNK_G)], src_v.at[X],
                        sem_i.at[X]),
                    pltpu.make_async_copy(
                        e_hbm.at[pl.ds(E + base, CHUNK_G)], dst_v.at[X],
                        sem_i.at[X]))

        def gather_desc(X):
            return pltpu.make_async_copy(
                hw_hbm.at[src_v.at[X]], rows_v.at[X], sem_g.at[X])

        def scatter_start(X):
            pltpu.async_copy(
                rows_v.at[X], acc.at[dst_v.at[X]], sem_s.at[X], add=True)
            pltpu.async_copy(
                exrow_v.at[X], acc2.at[dst_v.at[X]], sem_c.at[X], add=True)

        def scatter_wait(X):
            pltpu.make_async_copy(
                rows_v.at[X], acc.at[dst_v.at[X]], sem_s.at[X]).wait()
            pltpu.make_async_copy(
                exrow_v.at[X], acc2.at[dst_v.at[X]], sem_c.at[X]).wait()

        @pl.loop(0, rounds)
        def _(j):
            base0 = tile_base + j * (SLOTS * CHUNK_G)
            for X in range(SLOTS):
                @pl.when(j > 0)
                def _():
                    scatter_wait(X)
                d1, d2 = idx_descs(X, base0 + X * CHUNK_G)
                d1.start()
                d2.start()
            for X in range(SLOTS):
                d1, d2 = idx_descs(X, base0 + X * CHUNK_G)
                d1.wait()
                d2.wait()
                gather_desc(X).start()
                for g in range(CHUNK_G // 16):
                    si = src_v[X, pl.ds(g * 16, 16)]
                    di = dst_v[X, pl.ds(g * 16, 16)]
                    av = plsc.load_gather(asrc_v, [si])
                    bv = plsc.load_gather(adst_v, [di])
                    e = av + bv
                    e = jnp.maximum(e, e * 0.2)
                    ex_v[X, pl.ds(g * 16, 16)] = jnp.exp(e)
            for X in range(SLOTS):
                gather_desc(X).wait()

                @plsc.parallel_loop(0, CHUNK_G, 1, unroll=4)
                def _(i):
                    spl = plsc.load_gather(
                        ex_v.at[X], [jnp.full((16,), i, jnp.int32)])
                    for cg in range(D_OUT // 16):
                        rows_v[X, i, pl.ds(cg * 16, 16)] = (
                            rows_v[X, i, pl.ds(cg * 16, 16)] * spl)
                    exrow_v[X, i, :] = spl

                scatter_start(X)

        for X in range(SLOTS):
            scatter_wait(X)
        plsc.subcore_barrier()
        pltpu.sync_copy(acc.at[pl.ds(row0, ROWS_PER_TILE)],
                        out_hbm.at[c].at[pl.ds(row0, ROWS_PER_TILE)])
        pltpu.sync_copy(acc2.at[pl.ds(row0, ROWS_PER_TILE)],
                        den_hbm.at[c].at[pl.ds(row0, ROWS_PER_TILE)])

    return k(hw, e_flat, aux, zeros, zeros16)


BR = 1024


def _tc_a(p, p2, x, w_1, b_s, w_2, a_s, a_d, b_lin):
    def body(p0_r, p1_r, c0_r, c1_r, x_r, w1_r, bs_r, w2_r, as_r, ad_r,
             blin_r, hw_r, skip_r, aux_r):
        agg = p0_r[...] + p1_r[...]
        cnt = c0_r[:, 0:1] + c1_r[:, 0:1]
        mean = agg / jnp.maximum(cnt, 1.0)
        lhs = jnp.concatenate([mean, x_r[...]], axis=1)
        h = jnp.dot(lhs, w1_r[...]) + bs_r[...]
        h = jnp.maximum(h, 0.0)
        hs = jnp.dot(h, w2_r[...])
        hw = hs[:, :D_OUT]
        skip_r[...] = hs[:, D_OUT:] + blin_r[...]
        av = lax.dot_general(as_r[...], hw, (((1,), (1,)), ((), ())))
        dv = lax.dot_general(ad_r[...], hw, (((1,), (1,)), ((), ())))
        hw_r[...] = hw
        aux_r[...] = jnp.concatenate(
            [av, dv, jnp.zeros((14, BR), jnp.float32)], axis=0)

    full = lambda shp: pl.BlockSpec(shp, lambda i: (0,) * len(shp))
    return pl.pallas_call(
        body,
        grid=(pl.cdiv(N, BR),),
        in_specs=[
            pl.BlockSpec((None, BR, D_IN), lambda i: (0, i, 0)),
            pl.BlockSpec((None, BR, D_IN), lambda i: (1, i, 0)),
            pl.BlockSpec((None, BR, 16), lambda i: (0, i, 0)),
            pl.BlockSpec((None, BR, 16), lambda i: (1, i, 0)),
            pl.BlockSpec((BR, D_IN), lambda i: (i, 0)),
            full((2 * D_IN, D_HID)),
            full((1, D_HID)),
            full((D_HID, 2 * D_OUT)),
            full((1, D_OUT)),
            full((1, D_OUT)),
            full((1, D_OUT)),
        ],
        out_specs=[
            pl.BlockSpec((BR, D_OUT), lambda i: (i, 0)),
            pl.BlockSpec((BR, D_OUT), lambda i: (i, 0)),
            pl.BlockSpec((16, BR), lambda i: (0, i)),
        ],
        out_shape=[
            jax.ShapeDtypeStruct((N, D_OUT), jnp.float32),
            jax.ShapeDtypeStruct((N, D_OUT), jnp.float32),
            jax.ShapeDtypeStruct((16, N_ACC), jnp.float32),
        ],
    )(p, p, p2, p2, x, w_1, b_s, w_2, a_s, a_d, b_lin)


def _tc_b(q, q2, skip, b_g):
    def body(q0_r, q1_r, d0_r, d1_r, skip_r, bg_r, out_r):
        s = q0_r[...] + q1_r[...]
        denom = jnp.maximum(d0_r[:, 0:1] + d1_r[:, 0:1], 1e-16)
        out_r[...] = s / denom + bg_r[...] + skip_r[...]

    return pl.pallas_call(
        body,
        grid=(pl.cdiv(N, BR),),
        in_specs=[
            pl.BlockSpec((None, BR, D_OUT), lambda i: (0, i, 0)),
            pl.BlockSpec((None, BR, D_OUT), lambda i: (1, i, 0)),
            pl.BlockSpec((None, BR, 16), lambda i: (0, i, 0)),
            pl.BlockSpec((None, BR, 16), lambda i: (1, i, 0)),
            pl.BlockSpec((BR, D_OUT), lambda i: (i, 0)),
            pl.BlockSpec((1, D_OUT), lambda i: (0, 0)),
        ],
        out_specs=pl.BlockSpec((BR, D_OUT), lambda i: (i, 0)),
        out_shape=jax.ShapeDtypeStruct((N, D_OUT), jnp.float32),
    )(q, q, q2, q2, skip, b_g)


def kernel(x, edge_index, W_sage_l, W_sage_r, b_sage, W_gat, att_src,
           att_dst, b_gat, W_lin, b_lin):
    e_flat = edge_index.astype(jnp.int32).reshape(2 * E)
    zeros = jnp.zeros((N_ACC, D_IN), jnp.float32)
    zeros16 = jnp.zeros((N_ACC, 16), jnp.float32)
    w_1 = jnp.concatenate([W_sage_l, W_sage_r], axis=0)
    w_2 = jnp.concatenate([W_gat, W_lin], axis=1)

    p, p2 = _sage_sc(x, e_flat, zeros, zeros16)
    hw, skip, aux = _tc_a(
        p, p2, x, w_1, b_sage.reshape(1, -1), w_2,
        att_src.reshape(1, -1), att_dst.reshape(1, -1),
        b_lin.reshape(1, -1))
    q, q2 = _gat_sc(hw, e_flat, aux, zeros, zeros16)
    out = _tc_b(q, q2, skip, b_gat.reshape(1, -1))
    return out
